# Initial kernel scaffold; baseline (speedup 1.0000x reference)
import jax
import jax.numpy as jnp
from jax.experimental import pallas as pl


def kernel(x, person_emb):
    raise NotImplementedError("write your pallas kernel here")


# speedup vs baseline: 5.0438x; 5.0438x over previous
import jax
import jax.numpy as jnp
from jax.experimental import pallas as pl


def kernel(x, person_emb):
    B, T, P, D = x.shape
    BT = B * T
    ROW = P * D
    BT_BLK = 400

    emb_row = person_emb.reshape(1, ROW)

    def body(emb_ref, o_ref):
        o_ref[...] = jnp.broadcast_to(emb_ref[...], (BT_BLK, ROW))

    out = pl.pallas_call(
        body,
        grid=(BT // BT_BLK,),
        in_specs=[pl.BlockSpec((1, ROW), lambda i: (0, 0))],
        out_specs=pl.BlockSpec((BT_BLK, ROW), lambda i: (i, 0)),
        out_shape=jax.ShapeDtypeStruct((BT, ROW), person_emb.dtype),
    )(emb_row)
    return out.reshape(B, T, P, D)


# speedup vs baseline: 6.4489x; 1.2786x over previous
import jax
import jax.numpy as jnp
from jax.experimental import pallas as pl


def kernel(x, person_emb):
    B, T, P, D = x.shape
    T_BLK = 100

    def body(emb_ref, o_ref):
        o_ref[...] = jnp.broadcast_to(emb_ref[...], (1, T_BLK, P, D))

    return pl.pallas_call(
        body,
        grid=(B, T // T_BLK),
        in_specs=[pl.BlockSpec((P, D), lambda i, j: (0, 0))],
        out_specs=pl.BlockSpec((1, T_BLK, P, D), lambda i, j: (i, j, 0, 0)),
        out_shape=jax.ShapeDtypeStruct((B, T, P, D), person_emb.dtype),
    )(person_emb)


# speedup vs baseline: 6.7070x; 1.0400x over previous
import jax
import jax.numpy as jnp
from jax.experimental import pallas as pl
from jax.experimental.pallas import tpu as pltpu


def kernel(x, person_emb):
    B, T, P, D = x.shape
    T_BLK = 100
    NJ = T // T_BLK

    def body(emb_ref, o_ref, buf, sem):
        buf[...] = jnp.broadcast_to(emb_ref[...][None, :, :], (T_BLK, P, D))
        for i in range(B):
            for j in range(NJ):
                pltpu.make_async_copy(
                    buf, o_ref.at[i, pl.ds(j * T_BLK, T_BLK)], sem
                ).start()
        for _ in range(B * NJ):
            pltpu.make_async_copy(
                buf, o_ref.at[0, pl.ds(0, T_BLK)], sem
            ).wait()

    return pl.pallas_call(
        body,
        in_specs=[pl.BlockSpec(memory_space=pltpu.VMEM)],
        out_specs=pl.BlockSpec(memory_space=pl.ANY),
        out_shape=jax.ShapeDtypeStruct((B, T, P, D), person_emb.dtype),
        scratch_shapes=[
            pltpu.VMEM((T_BLK, P, D), person_emb.dtype),
            pltpu.SemaphoreType.DMA,
        ],
    )(person_emb)
